# baseline (speedup 1.0000x reference)
import functools

import jax
import jax.numpy as jnp
from jax import lax
from jax.experimental import pallas as pl
from jax.experimental.pallas import tpu as pltpu
from jax.experimental.pallas import tpu_sc as plsc

EPS = 1e-12


GATHER_WINDOW = 128


def _sc_gather_body(table_hbm, i_hbm, o_hbm):
    def body(i_vmem, o_vmem):
        pltpu.sync_copy(table_hbm.at[i_vmem.at[0]], o_vmem)

    num_windows = i_hbm.shape[0]
    pltpu.emit_pipeline(
        body,
        grid=(num_windows,),
        in_specs=[pl.BlockSpec((1, GATHER_WINDOW), index_map=lambda w: (w, 0))],
        out_specs=[pl.BlockSpec((GATHER_WINDOW, table_hbm.shape[1]),
                                index_map=lambda w: (w, 0))],
        core_axis_name=("c", "s"),
        dimension_semantics=(pltpu.PARALLEL,),
    )(i_hbm, o_hbm)


def _sc_gather(table, idx):
    n = idx.shape[0]
    idx2 = idx.reshape(n // GATHER_WINDOW, GATHER_WINDOW)
    mesh = plsc.VectorSubcoreMesh(core_axis_name="c", subcore_axis_name="s")
    kern = pl.kernel(
        _sc_gather_body,
        out_type=jax.ShapeDtypeStruct((n, table.shape[1]), table.dtype),
        mesh=mesh,
    )
    return kern(table, idx2)


ROW_BLOCK = 8192
CHUNK_BLOCKS = [5, 4, 4, 4, 4, 4]


HALVES = 2


def _tc_ln_body(g_ref, pid_ref, sid_ref, tbl_ref, sc_ref, bi_ref, o_ref):
    r = g_ref.shape[0]
    k = tbl_ref.shape[0]
    max_pos = k - 2
    tbl = tbl_ref[...]
    rh = r // HALVES
    for hf in range(HALVES):
        rows = pl.ds(hf * rh, rh)
        x = g_ref[rows, :]
        pid = pid_ref[0, :, pl.ds(hf * rh, rh)]
        sid = sid_ref[0, :, pl.ds(hf * rh, rh)]
        iota_p = lax.broadcasted_iota(jnp.int32, (max_pos, rh), 0)
        iota_s = lax.broadcasted_iota(jnp.int32, (2, rh), 0)
        oh = jnp.concatenate(
            [(iota_p == pid).astype(jnp.bfloat16),
             (iota_s == sid).astype(jnp.bfloat16)], axis=0)
        posegv = lax.dot_general(
            oh, tbl,
            dimension_numbers=(((0,), (0,)), ((), ())),
            preferred_element_type=jnp.float32,
        )
        x = x + posegv
        m = jnp.mean(x, axis=-1, keepdims=True)
        d = x - m
        v = jnp.mean(d * d, axis=-1, keepdims=True)
        normed = d * lax.rsqrt(v + EPS)
        o_ref[rows, :] = normed * sc_ref[...] + bi_ref[...]


def _tc_ln_chunk_body(g_ref, pid_ref, sid_ref, tbl_ref, sc_ref, bi_ref,
                      prev_ref, o_ref):
    del prev_ref
    _tc_ln_body(g_ref, pid_ref, sid_ref, tbl_ref, sc_ref, bi_ref, o_ref)


def _tc_ln_chunk(gathered_k, pos_ids_k, seg_ids_k, tbl, ln_scale, ln_bias,
                 prev_out, block_off, n):
    ck, h = gathered_k.shape
    grid = (ck // ROW_BLOCK,)
    return pl.pallas_call(
        _tc_ln_chunk_body,
        grid=grid,
        in_specs=[
            pl.BlockSpec((ROW_BLOCK, h), lambda i: (i, 0)),
            pl.BlockSpec((1, 1, ROW_BLOCK), lambda i: (i, 0, 0)),
            pl.BlockSpec((1, 1, ROW_BLOCK), lambda i: (i, 0, 0)),
            pl.BlockSpec(tbl.shape, lambda i: (0, 0)),
            pl.BlockSpec((1, h), lambda i: (0, 0)),
            pl.BlockSpec((1, h), lambda i: (0, 0)),
            pl.BlockSpec(memory_space=pltpu.HBM),
        ],
        out_specs=pl.BlockSpec((ROW_BLOCK, h),
                               lambda i, _o=block_off: (i + _o, 0)),
        out_shape=jax.ShapeDtypeStruct((n, h), jnp.float32),
        input_output_aliases={6: 0},
        compiler_params=pltpu.CompilerParams(
            dimension_semantics=("arbitrary",),
        ),
    )(gathered_k, pos_ids_k, seg_ids_k, tbl, ln_scale, ln_bias, prev_out)


def _tc_ln_first(gathered_k, pos_ids_k, seg_ids_k, tbl, ln_scale, ln_bias, n):
    ck, h = gathered_k.shape
    grid = (ck // ROW_BLOCK,)
    return pl.pallas_call(
        _tc_ln_body,
        grid=grid,
        in_specs=[
            pl.BlockSpec((ROW_BLOCK, h), lambda i: (i, 0)),
            pl.BlockSpec((1, 1, ROW_BLOCK), lambda i: (i, 0, 0)),
            pl.BlockSpec((1, 1, ROW_BLOCK), lambda i: (i, 0, 0)),
            pl.BlockSpec(tbl.shape, lambda i: (0, 0)),
            pl.BlockSpec((1, h), lambda i: (0, 0)),
            pl.BlockSpec((1, h), lambda i: (0, 0)),
        ],
        out_specs=pl.BlockSpec((ROW_BLOCK, h), lambda i: (i, 0)),
        out_shape=jax.ShapeDtypeStruct((n, h), jnp.float32),
        compiler_params=pltpu.CompilerParams(
            dimension_semantics=("arbitrary",),
        ),
    )(gathered_k, pos_ids_k, seg_ids_k, tbl, ln_scale, ln_bias)


def kernel(input_ids, token_type_ids, position_ids, tok_emb, pos_emb, seg_emb,
           ln_scale, ln_bias):
    b, l = input_ids.shape
    h = tok_emb.shape[1]
    n = b * l
    ids = input_ids.reshape(-1).astype(jnp.int32)
    pids = position_ids.reshape(n // ROW_BLOCK, 1, ROW_BLOCK).astype(jnp.int32)
    sids = token_type_ids.reshape(n // ROW_BLOCK, 1, ROW_BLOCK).astype(jnp.int32)

    tbl = jnp.concatenate([pos_emb, seg_emb], axis=0).astype(jnp.bfloat16)
    scale = ln_scale.reshape(1, h)
    bias = ln_bias.reshape(1, h)

    offs = [0]
    for nb in CHUNK_BLOCKS:
        offs.append(offs[-1] + nb)
    assert offs[-1] * ROW_BLOCK == n
    gs = [_sc_gather(tok_emb, ids[offs[k] * ROW_BLOCK:offs[k + 1] * ROW_BLOCK])
          for k in range(len(CHUNK_BLOCKS))]
    out = _tc_ln_first(gs[0], pids[:offs[1]], sids[:offs[1]],
                       tbl, scale, bias, n)
    for k in range(1, len(CHUNK_BLOCKS)):
        out = _tc_ln_chunk(
            gs[k], pids[offs[k]:offs[k + 1]], sids[offs[k]:offs[k + 1]],
            tbl, scale, bias, out, offs[k], n)
    return out.reshape(b, l, h)


# speedup vs baseline: 1.0615x; 1.0615x over previous
import functools

import jax
import jax.numpy as jnp
from jax import lax
from jax.experimental import pallas as pl
from jax.experimental.pallas import tpu as pltpu
from jax.experimental.pallas import tpu_sc as plsc

EPS = 1e-12


GATHER_WINDOW = 128


def _sc_gather_body(win_off, num_windows, table_hbm, i_hbm, o_hbm):
    def body(i_vmem, o_vmem):
        pltpu.sync_copy(table_hbm.at[i_vmem.at[0]], o_vmem)

    pltpu.emit_pipeline(
        body,
        grid=(num_windows,),
        in_specs=[pl.BlockSpec((1, GATHER_WINDOW),
                               index_map=lambda w, _o=win_off: (w + _o, 0))],
        out_specs=[pl.BlockSpec((GATHER_WINDOW, table_hbm.shape[1]),
                                index_map=lambda w: (w, 0))],
        core_axis_name=("c", "s"),
        dimension_semantics=(pltpu.PARALLEL,),
    )(i_hbm, o_hbm)


def _sc_gather(table, idx2, row_off, nrows):
    mesh = plsc.VectorSubcoreMesh(core_axis_name="c", subcore_axis_name="s")
    kern = pl.kernel(
        functools.partial(_sc_gather_body, row_off // GATHER_WINDOW,
                          nrows // GATHER_WINDOW),
        out_type=jax.ShapeDtypeStruct((nrows, table.shape[1]), table.dtype),
        mesh=mesh,
    )
    return kern(table, idx2)


ROW_BLOCK = 8192
CHUNK_BLOCKS = [5, 5, 5, 5, 5]


HALVES = 2


def _tc_ln_body(g_ref, pid_ref, sid_ref, tbl_ref, sc_ref, bi_ref, o_ref):
    r = g_ref.shape[0]
    k = tbl_ref.shape[0]
    max_pos = k - 2
    tbl = tbl_ref[...]
    rh = r // HALVES
    for hf in range(HALVES):
        rows = pl.ds(hf * rh, rh)
        x = g_ref[rows, :]
        pid = pid_ref[0, :, pl.ds(hf * rh, rh)]
        sid = sid_ref[0, :, pl.ds(hf * rh, rh)]
        iota_p = lax.broadcasted_iota(jnp.int32, (max_pos, rh), 0)
        iota_s = lax.broadcasted_iota(jnp.int32, (2, rh), 0)
        oh = jnp.concatenate(
            [(iota_p == pid).astype(jnp.bfloat16),
             (iota_s == sid).astype(jnp.bfloat16)], axis=0)
        posegv = lax.dot_general(
            oh, tbl,
            dimension_numbers=(((0,), (0,)), ((), ())),
            preferred_element_type=jnp.float32,
        )
        x = x + posegv
        m = jnp.mean(x, axis=-1, keepdims=True)
        d = x - m
        v = jnp.mean(d * d, axis=-1, keepdims=True)
        normed = d * lax.rsqrt(v + EPS)
        o_ref[rows, :] = normed * sc_ref[...] + bi_ref[...]


def _tc_ln_chunk_body(g_ref, pid_ref, sid_ref, tbl_ref, sc_ref, bi_ref,
                      prev_ref, o_ref):
    del prev_ref
    _tc_ln_body(g_ref, pid_ref, sid_ref, tbl_ref, sc_ref, bi_ref, o_ref)


def _tc_ln_chunk(gathered_k, pos_ids, seg_ids, tbl, ln_scale, ln_bias,
                 prev_out, block_off, n):
    ck, h = gathered_k.shape
    grid = (ck // ROW_BLOCK,)
    body = _tc_ln_chunk_body if prev_out is not None else _tc_ln_body
    in_specs = [
        pl.BlockSpec((ROW_BLOCK, h), lambda i: (i, 0)),
        pl.BlockSpec((1, 1, ROW_BLOCK),
                     lambda i, _o=block_off: (i + _o, 0, 0)),
        pl.BlockSpec((1, 1, ROW_BLOCK),
                     lambda i, _o=block_off: (i + _o, 0, 0)),
        pl.BlockSpec(tbl.shape, lambda i: (0, 0)),
        pl.BlockSpec((1, h), lambda i: (0, 0)),
        pl.BlockSpec((1, h), lambda i: (0, 0)),
    ]
    args = [gathered_k, pos_ids, seg_ids, tbl, ln_scale, ln_bias]
    aliases = {}
    if prev_out is not None:
        in_specs.append(pl.BlockSpec(memory_space=pltpu.HBM))
        args.append(prev_out)
        aliases = {6: 0}
    return pl.pallas_call(
        body,
        grid=grid,
        in_specs=in_specs,
        out_specs=pl.BlockSpec((ROW_BLOCK, h),
                               lambda i, _o=block_off: (i + _o, 0)),
        out_shape=jax.ShapeDtypeStruct((n, h), jnp.float32),
        input_output_aliases=aliases,
        compiler_params=pltpu.CompilerParams(
            dimension_semantics=("arbitrary",),
        ),
    )(*args)


def kernel(input_ids, token_type_ids, position_ids, tok_emb, pos_emb, seg_emb,
           ln_scale, ln_bias):
    b, l = input_ids.shape
    h = tok_emb.shape[1]
    n = b * l
    ids = input_ids.reshape(-1).astype(jnp.int32)
    pids = position_ids.reshape(n // ROW_BLOCK, 1, ROW_BLOCK).astype(jnp.int32)
    sids = token_type_ids.reshape(n // ROW_BLOCK, 1, ROW_BLOCK).astype(jnp.int32)

    tbl = jnp.concatenate([pos_emb, seg_emb], axis=0).astype(jnp.bfloat16)
    scale = ln_scale.reshape(1, h)
    bias = ln_bias.reshape(1, h)

    ids2 = ids.reshape(n // GATHER_WINDOW, GATHER_WINDOW)
    offs = [0]
    for nb in CHUNK_BLOCKS:
        offs.append(offs[-1] + nb)
    assert offs[-1] * ROW_BLOCK == n
    gs = [_sc_gather(tok_emb, ids2, offs[k] * ROW_BLOCK,
                     CHUNK_BLOCKS[k] * ROW_BLOCK)
          for k in range(len(CHUNK_BLOCKS))]
    out = None
    for k in range(len(CHUNK_BLOCKS)):
        out = _tc_ln_chunk(gs[k], pids, sids, tbl, scale, bias,
                           out, offs[k], n)
    return out.reshape(b, l, h)


# speedup vs baseline: 1.0650x; 1.0033x over previous
import functools

import jax
import jax.numpy as jnp
from jax import lax
from jax.experimental import pallas as pl
from jax.experimental.pallas import tpu as pltpu
from jax.experimental.pallas import tpu_sc as plsc

EPS = 1e-12


GATHER_WINDOW = 128


def _sc_gather_body(win_off, num_windows, table_hbm, i_hbm, o_hbm):
    def body(i_vmem, o_vmem):
        pltpu.sync_copy(table_hbm.at[i_vmem.at[0]], o_vmem)

    pltpu.emit_pipeline(
        body,
        grid=(num_windows,),
        in_specs=[pl.BlockSpec((1, GATHER_WINDOW),
                               index_map=lambda w, _o=win_off: (w + _o, 0))],
        out_specs=[pl.BlockSpec((GATHER_WINDOW, table_hbm.shape[1]),
                                index_map=lambda w: (w, 0))],
        core_axis_name=("c", "s"),
        dimension_semantics=(pltpu.PARALLEL,),
    )(i_hbm, o_hbm)


def _sc_gather(table, idx2, row_off, nrows):
    mesh = plsc.VectorSubcoreMesh(core_axis_name="c", subcore_axis_name="s")
    kern = pl.kernel(
        functools.partial(_sc_gather_body, row_off // GATHER_WINDOW,
                          nrows // GATHER_WINDOW),
        out_type=jax.ShapeDtypeStruct((nrows, table.shape[1]), table.dtype),
        mesh=mesh,
    )
    return kern(table, idx2)


ROW_BLOCK = 8192
CHUNK_BLOCKS = [5, 5, 5, 5, 5]


HALVES = 4


def _tc_ln_body(g_ref, pid_ref, sid_ref, tbl_ref, sc_ref, bi_ref, o_ref):
    r = g_ref.shape[0]
    k = tbl_ref.shape[0]
    max_pos = k - 2
    tbl = tbl_ref[...]
    rh = r // HALVES
    for hf in range(HALVES):
        rows = pl.ds(hf * rh, rh)
        x = g_ref[rows, :]
        pid = pid_ref[0, :, pl.ds(hf * rh, rh)]
        sid = sid_ref[0, :, pl.ds(hf * rh, rh)]
        iota_p = lax.broadcasted_iota(jnp.int32, (max_pos, rh), 0)
        iota_s = lax.broadcasted_iota(jnp.int32, (2, rh), 0)
        oh = jnp.concatenate(
            [(iota_p == pid).astype(jnp.bfloat16),
             (iota_s == sid).astype(jnp.bfloat16)], axis=0)
        posegv = lax.dot_general(
            oh, tbl,
            dimension_numbers=(((0,), (0,)), ((), ())),
            preferred_element_type=jnp.float32,
        )
        x = x + posegv
        m = jnp.mean(x, axis=-1, keepdims=True)
        d = x - m
        v = jnp.mean(d * d, axis=-1, keepdims=True)
        normed = d * lax.rsqrt(v + EPS)
        o_ref[rows, :] = normed * sc_ref[...] + bi_ref[...]


def _tc_ln_chunk_body(g_ref, pid_ref, sid_ref, tbl_ref, sc_ref, bi_ref,
                      prev_ref, o_ref):
    del prev_ref
    _tc_ln_body(g_ref, pid_ref, sid_ref, tbl_ref, sc_ref, bi_ref, o_ref)


def _tc_ln_chunk(gathered_k, pos_ids, seg_ids, tbl, ln_scale, ln_bias,
                 prev_out, block_off, n):
    ck, h = gathered_k.shape
    grid = (ck // ROW_BLOCK,)
    body = _tc_ln_chunk_body if prev_out is not None else _tc_ln_body
    in_specs = [
        pl.BlockSpec((ROW_BLOCK, h), lambda i: (i, 0)),
        pl.BlockSpec((1, 1, ROW_BLOCK),
                     lambda i, _o=block_off: (i + _o, 0, 0)),
        pl.BlockSpec((1, 1, ROW_BLOCK),
                     lambda i, _o=block_off: (i + _o, 0, 0)),
        pl.BlockSpec(tbl.shape, lambda i: (0, 0)),
        pl.BlockSpec((1, h), lambda i: (0, 0)),
        pl.BlockSpec((1, h), lambda i: (0, 0)),
    ]
    args = [gathered_k, pos_ids, seg_ids, tbl, ln_scale, ln_bias]
    aliases = {}
    if prev_out is not None:
        in_specs.append(pl.BlockSpec(memory_space=pltpu.HBM))
        args.append(prev_out)
        aliases = {6: 0}
    return pl.pallas_call(
        body,
        grid=grid,
        in_specs=in_specs,
        out_specs=pl.BlockSpec((ROW_BLOCK, h),
                               lambda i, _o=block_off: (i + _o, 0)),
        out_shape=jax.ShapeDtypeStruct((n, h), jnp.float32),
        input_output_aliases=aliases,
        compiler_params=pltpu.CompilerParams(
            dimension_semantics=("arbitrary",),
        ),
    )(*args)


def kernel(input_ids, token_type_ids, position_ids, tok_emb, pos_emb, seg_emb,
           ln_scale, ln_bias):
    b, l = input_ids.shape
    h = tok_emb.shape[1]
    n = b * l
    ids = input_ids.reshape(-1).astype(jnp.int32)
    pids = position_ids.reshape(n // ROW_BLOCK, 1, ROW_BLOCK).astype(jnp.int32)
    sids = token_type_ids.reshape(n // ROW_BLOCK, 1, ROW_BLOCK).astype(jnp.int32)

    tbl = jnp.concatenate([pos_emb, seg_emb], axis=0).astype(jnp.bfloat16)
    scale = ln_scale.reshape(1, h)
    bias = ln_bias.reshape(1, h)

    ids2 = ids.reshape(n // GATHER_WINDOW, GATHER_WINDOW)
    offs = [0]
    for nb in CHUNK_BLOCKS:
        offs.append(offs[-1] + nb)
    assert offs[-1] * ROW_BLOCK == n
    gs = [_sc_gather(tok_emb, ids2, offs[k] * ROW_BLOCK,
                     CHUNK_BLOCKS[k] * ROW_BLOCK)
          for k in range(len(CHUNK_BLOCKS))]
    out = None
    for k in range(len(CHUNK_BLOCKS)):
        out = _tc_ln_chunk(gs[k], pids, sids, tbl, scale, bias,
                           out, offs[k], n)
    return out.reshape(b, l, h)
